# baseline (speedup 1.0000x reference)
import jax
import jax.numpy as jnp
from jax import lax
from jax.experimental import pallas as pl
from jax.experimental.pallas import tpu as pltpu
from jax.experimental.pallas import tpu_sc as plsc

_NC, _NS, _L = 2, 16, 16
_NW = _NC * _NS

_K_SC = 160
_CPW = _K_SC // _NW
_CC = 105


def _tc_step(lab_ref, x_ref, out_ref):
    x = x_ref[...]
    s = jnp.sum(x, axis=1)
    d = s - 1.0
    part = jnp.sum(d * d)
    lab = lab_ref[0, :]
    c_idx = (jax.lax.broadcasted_iota(jnp.int32, s.shape, 0)
             + pl.program_id(0) * _CC)
    corr = jnp.sum(jnp.where(lab[None, :] == c_idx, 99.0 - 18.0 * s, 0.0))
    out_ref[...] = (part + corr).reshape(1, 1, 1)


def _sc_body(x_hbm, lab_hbm, out_hbm, lab_v, buf0, buf1, acc_v,
             sem0, sem1):
    C = x_hbm.shape[0]
    T, B = 64, 256
    c_tc = C - _K_SC
    wid = lax.axis_index("s") * _NC + lax.axis_index("c")
    base = c_tc + wid * _CPW
    pltpu.sync_copy(lab_hbm, lab_v)
    bufs = (buf0, buf1)
    sems = (sem0, sem1)
    copies = [None, None]
    copies[0] = pltpu.make_async_copy(x_hbm.at[base], buf0, sem0)
    copies[0].start()
    total = jnp.zeros((_L,), jnp.float32)
    for j in range(_CPW):
        buf = bufs[j % 2]
        copies[j % 2].wait()
        if j + 1 < _CPW:
            copies[(j + 1) % 2] = pltpu.make_async_copy(
                x_hbm.at[base + j + 1], bufs[(j + 1) % 2], sems[(j + 1) % 2])
            copies[(j + 1) % 2].start()
        c_cur = base + j
        for k in range(B // _L):
            off = k * _L

            def _tstep(t, a):
                return a + buf[pl.ds(t * B + off, _L)]

            s = lax.fori_loop(0, T, _tstep, jnp.zeros((_L,), jnp.float32))
            d = s - 1.0
            lv = lab_v[pl.ds(off, _L)]
            total = total + d * d + jnp.where(
                lv == c_cur, 99.0 - 18.0 * s, 0.0)
    acc_v[...] = total
    pltpu.sync_copy(acc_v, out_hbm.at[wid])


def kernel(outputs, labels):
    B, C, H, W, T = outputs.shape
    xt = jnp.transpose(outputs.reshape(B, C, T), (1, 2, 0))
    c_tc = C - _K_SC
    n_steps = c_tc // _CC
    lab2 = labels.reshape(1, B)
    tc_out = pl.pallas_call(
        _tc_step,
        grid=(n_steps,),
        in_specs=[
            pl.BlockSpec((1, B), lambda i: (0, 0)),
            pl.BlockSpec((_CC, T, B), lambda i: (i, 0, 0)),
        ],
        out_specs=pl.BlockSpec((1, 1, 1), lambda i: (i, 0, 0)),
        out_shape=jax.ShapeDtypeStruct((n_steps, 1, 1), jnp.float32),
        compiler_params=pltpu.CompilerParams(
            dimension_semantics=("parallel",)),
    )(lab2, xt)

    x_sc = xt.reshape(C, T * B)
    mesh = plsc.VectorSubcoreMesh(
        core_axis_name="c", subcore_axis_name="s",
        num_cores=_NC, num_subcores=_NS)
    sc_out = pl.kernel(
        _sc_body,
        out_type=jax.ShapeDtypeStruct((_NW, _L), jnp.float32),
        mesh=mesh,
        scratch_types=[
            pltpu.VMEM((B,), jnp.int32),
            pltpu.VMEM((T * B,), jnp.float32),
            pltpu.VMEM((T * B,), jnp.float32),
            pltpu.VMEM((_L,), jnp.float32),
            pltpu.SemaphoreType.DMA,
            pltpu.SemaphoreType.DMA,
        ],
    )(x_sc, labels)

    return (0.5 / T) * (jnp.sum(tc_out) + jnp.sum(sc_out))


# speedup vs baseline: 2.3885x; 2.3885x over previous
import jax
import jax.numpy as jnp
from jax import lax
from jax.experimental import pallas as pl
from jax.experimental.pallas import tpu as pltpu
from jax.experimental.pallas import tpu_sc as plsc

_NC, _NS, _L = 2, 16, 16
_NW = _NC * _NS

_K_SC = 160
_CPW = _K_SC // _NW
_CC = 105


def _tc_step(lab_ref, x_ref, out_ref):
    x = x_ref[...]
    s = jnp.sum(x, axis=1)
    d = s - 1.0
    part = jnp.sum(d * d)
    lab = lab_ref[0, :]
    c_idx = (jax.lax.broadcasted_iota(jnp.int32, s.shape, 0)
             + pl.program_id(0) * _CC)
    corr = jnp.sum(jnp.where(lab[None, :] == c_idx, 99.0 - 18.0 * s, 0.0))
    out_ref[...] = (part + corr).reshape(1, 1, 1)


def _sc_body(x_hbm, lab_hbm, out_hbm, lab_v, buf0, buf1, acc_v,
             sem0, sem1):
    T, B = 64, 256
    C = x_hbm.shape[0] // T
    c_tc = C - _K_SC
    nk = B // _L
    wid = lax.axis_index("s") * _NC + lax.axis_index("c")
    base = c_tc + wid * _CPW
    pltpu.sync_copy(lab_hbm, lab_v)
    bufs = (buf0, buf1)
    sems = (sem0, sem1)
    copies = [None, None]
    copies[0] = pltpu.make_async_copy(
        x_hbm.at[pl.ds(base * T, T)], buf0, sem0)
    copies[0].start()
    total = jnp.zeros((_L,), jnp.float32)
    for j in range(_CPW):
        buf = bufs[j % 2]
        copies[j % 2].wait()
        if j + 1 < _CPW:
            copies[(j + 1) % 2] = pltpu.make_async_copy(
                x_hbm.at[pl.ds((base + j + 1) * T, T)],
                bufs[(j + 1) % 2], sems[(j + 1) % 2])
            copies[(j + 1) % 2].start()
        c_cur = base + j

        def _t_outer(i, carry):
            accs = list(carry)
            for t_in in range(8):
                t = i * 8 + t_in
                for k in range(nk):
                    accs[k] = accs[k] + buf[t, pl.ds(k * _L, _L)]
            return tuple(accs)

        s_chunks = lax.fori_loop(
            0, T // 8, _t_outer,
            tuple(jnp.zeros((_L,), jnp.float32) for _ in range(nk)))
        for k in range(nk):
            s = s_chunks[k]
            d = s - 1.0
            lv = lab_v[pl.ds(k * _L, _L)]
            total = total + d * d + jnp.where(
                lv == c_cur, 99.0 - 18.0 * s, 0.0)
    acc_v[...] = total
    pltpu.sync_copy(acc_v, out_hbm.at[wid])


def kernel(outputs, labels):
    B, C, H, W, T = outputs.shape
    xt = jnp.transpose(outputs.reshape(B, C, T), (1, 2, 0))
    c_tc = C - _K_SC
    n_steps = c_tc // _CC
    lab2 = labels.reshape(1, B)
    tc_out = pl.pallas_call(
        _tc_step,
        grid=(n_steps,),
        in_specs=[
            pl.BlockSpec((1, B), lambda i: (0, 0)),
            pl.BlockSpec((_CC, T, B), lambda i: (i, 0, 0)),
        ],
        out_specs=pl.BlockSpec((1, 1, 1), lambda i: (i, 0, 0)),
        out_shape=jax.ShapeDtypeStruct((n_steps, 1, 1), jnp.float32),
        compiler_params=pltpu.CompilerParams(
            dimension_semantics=("parallel",)),
    )(lab2, xt)

    x_sc = xt.reshape(C * T, B)
    mesh = plsc.VectorSubcoreMesh(
        core_axis_name="c", subcore_axis_name="s",
        num_cores=_NC, num_subcores=_NS)
    sc_out = pl.kernel(
        _sc_body,
        out_type=jax.ShapeDtypeStruct((_NW, _L), jnp.float32),
        mesh=mesh,
        scratch_types=[
            pltpu.VMEM((B,), jnp.int32),
            pltpu.VMEM((T, B), jnp.float32),
            pltpu.VMEM((T, B), jnp.float32),
            pltpu.VMEM((_L,), jnp.float32),
            pltpu.SemaphoreType.DMA,
            pltpu.SemaphoreType.DMA,
        ],
    )(x_sc, labels)

    return (0.5 / T) * (jnp.sum(tc_out) + jnp.sum(sc_out))


# speedup vs baseline: 2.4047x; 1.0068x over previous
import jax
import jax.numpy as jnp
from jax import lax
from jax.experimental import pallas as pl
from jax.experimental.pallas import tpu as pltpu
from jax.experimental.pallas import tpu_sc as plsc

_NC, _NS, _L = 2, 16, 16
_NW = _NC * _NS

_K_SC = 160
_CPW = _K_SC // _NW
_CC = 105


def _tc_step(lab_ref, x_ref, out_ref):
    x = x_ref[...]
    s = jnp.sum(x, axis=1)
    d = s - 1.0
    part = jnp.sum(d * d)
    lab = lab_ref[0, :]
    c_idx = (jax.lax.broadcasted_iota(jnp.int32, s.shape, 0)
             + pl.program_id(0) * _CC)
    corr = jnp.sum(jnp.where(lab[None, :] == c_idx, 99.0 - 18.0 * s, 0.0))
    out_ref[...] = (part + corr).reshape(1, 1, 1)


def _sc_body(x_hbm, lab_hbm, out_hbm, lab_v, buf0, buf1, acc_v,
             sem0, sem1):
    T, B = 64, 256
    C = x_hbm.shape[0] // T
    c_tc = C - _K_SC
    nk = B // _L
    wid = lax.axis_index("s") * _NC + lax.axis_index("c")
    base = c_tc + wid * _CPW
    pltpu.sync_copy(lab_hbm, lab_v)
    bufs = (buf0, buf1)
    sems = (sem0, sem1)
    copies = [None, None]
    copies[0] = pltpu.make_async_copy(
        x_hbm.at[pl.ds(base * T, T)], buf0, sem0)
    copies[0].start()
    total = jnp.zeros((_L,), jnp.float32)
    for j in range(_CPW):
        buf = bufs[j % 2]
        copies[j % 2].wait()
        if j + 1 < _CPW:
            copies[(j + 1) % 2] = pltpu.make_async_copy(
                x_hbm.at[pl.ds((base + j + 1) * T, T)],
                bufs[(j + 1) % 2], sems[(j + 1) % 2])
            copies[(j + 1) % 2].start()
        c_cur = base + j

        def _t_outer(i, carry):
            accs = list(carry)
            for t_in in range(8):
                t = i * 8 + t_in
                for k in range(nk):
                    accs[k] = accs[k] + buf[t, pl.ds(k * _L, _L)]
            return tuple(accs)

        s_chunks = lax.fori_loop(
            0, T // 8, _t_outer,
            tuple(jnp.zeros((_L,), jnp.float32) for _ in range(nk)))
        for k in range(nk):
            s = s_chunks[k]
            d = s - 1.0
            lv = lab_v[pl.ds(k * _L, _L)]
            total = total + d * d + jnp.where(
                lv == c_cur, 99.0 - 18.0 * s, 0.0)
    acc_v[...] = total
    pltpu.sync_copy(acc_v, out_hbm.at[wid])


def kernel(outputs, labels):
    B, C, H, W, T = outputs.shape
    xt = jnp.transpose(outputs.reshape(B, C, T), (1, 2, 0))
    c_tc = C - _K_SC
    n_steps = c_tc // _CC
    lab2 = labels.reshape(1, B)

    x_sc = xt.reshape(C * T, B)
    mesh = plsc.VectorSubcoreMesh(
        core_axis_name="c", subcore_axis_name="s",
        num_cores=_NC, num_subcores=_NS)
    sc_out = pl.kernel(
        _sc_body,
        out_type=jax.ShapeDtypeStruct((_NW, _L), jnp.float32),
        mesh=mesh,
        scratch_types=[
            pltpu.VMEM((B,), jnp.int32),
            pltpu.VMEM((T, B), jnp.float32),
            pltpu.VMEM((T, B), jnp.float32),
            pltpu.VMEM((_L,), jnp.float32),
            pltpu.SemaphoreType.DMA,
            pltpu.SemaphoreType.DMA,
        ],
    )(x_sc, labels)

    tc_out = pl.pallas_call(
        _tc_step,
        grid=(n_steps,),
        in_specs=[
            pl.BlockSpec((1, B), lambda i: (0, 0)),
            pl.BlockSpec((_CC, T, B), lambda i: (i, 0, 0)),
        ],
        out_specs=pl.BlockSpec((1, 1, 1), lambda i: (i, 0, 0)),
        out_shape=jax.ShapeDtypeStruct((n_steps, 1, 1), jnp.float32),
        compiler_params=pltpu.CompilerParams(
            dimension_semantics=("parallel",)),
    )(lab2, xt)

    return (0.5 / T) * (jnp.sum(tc_out) + jnp.sum(sc_out))


# speedup vs baseline: 4.5094x; 1.8752x over previous
import jax
import jax.numpy as jnp
from jax.experimental import pallas as pl
from jax.experimental.pallas import tpu as pltpu

_CC = 125


def _loss_step(lab_ref, x_ref, out_ref):
    x = x_ref[...]
    T = x.shape[1]
    s = jnp.sum(x, axis=1)
    d = s - 1.0
    part = jnp.sum(d * d)
    lab = lab_ref[0, :]
    c_idx = (jax.lax.broadcasted_iota(jnp.int32, s.shape, 0)
             + pl.program_id(0) * _CC)
    corr = jnp.sum(jnp.where(lab[None, :] == c_idx, 99.0 - 18.0 * s, 0.0))
    out_ref[...] = ((part + corr) * (0.5 / T)).reshape(1, 1, 1)


def kernel(outputs, labels):
    B, C, H, W, T = outputs.shape
    xt = jnp.transpose(outputs.reshape(B, C, T), (1, 2, 0))
    n_steps = C // _CC
    lab2 = labels.reshape(1, B)
    out = pl.pallas_call(
        _loss_step,
        grid=(n_steps,),
        in_specs=[
            pl.BlockSpec((1, B), lambda i: (0, 0)),
            pl.BlockSpec((_CC, T, B), lambda i: (i, 0, 0)),
        ],
        out_specs=pl.BlockSpec((1, 1, 1), lambda i: (i, 0, 0)),
        out_shape=jax.ShapeDtypeStruct((n_steps, 1, 1), jnp.float32),
        compiler_params=pltpu.CompilerParams(
            dimension_semantics=("parallel",)),
    )(lab2, xt)
    return jnp.sum(out)
